# baseline (speedup 1.0000x reference)
import functools

import jax
import jax.numpy as jnp
from jax import lax
from jax.experimental import pallas as pl
from jax.experimental.pallas import tpu as pltpu
from jax.experimental.pallas import tpu_sc as plsc

B = 16384
DU = 64
DI = 32
EPS = 1e-06

NC = 1
NS = 16
NW = NC * NS
BPW = B // NW
CH = 128
NCH = BPW // CH
HB = 256
L = 16


def _gather_body(users2, items2, gu_tab, gi_tab, gu_out, gi_out,
                 uidx, iidx, upidx, ipidx, upair, ipair, sem):
    wid = lax.axis_index("s") * NC + lax.axis_index("c")
    base = wid * BPW
    pltpu.sync_copy(users2.at[pl.ds(wid * NCH, NCH)], uidx)
    pltpu.sync_copy(items2.at[pl.ds(wid * NCH, NCH)], iidx)
    for c in range(NCH):
        for k in range(CH // L):
            s = pl.ds(k * L, L)
            upidx[c, s] = uidx[c, s] >> 1
            ipidx[c, s] = iidx[c, s] >> 2
    for h in range(BPW // HB):
        copies = []
        for c in range(2):
            cc = h * 2 + c
            copies.append(pltpu.async_copy(
                gu_tab.at[upidx.at[cc]], upair.at[pl.ds(c * CH, CH)], sem))
            copies.append(pltpu.async_copy(
                gi_tab.at[ipidx.at[cc]], ipair.at[pl.ds(c * CH, CH)], sem))
        for cp in copies:
            cp.wait()
        pltpu.sync_copy(upair, gu_out.at[pl.ds(base + h * HB, HB)])
        pltpu.sync_copy(ipair, gi_out.at[pl.ds(base + h * HB, HB)])


_sc_gather = functools.partial(
    pl.kernel,
    mesh=plsc.VectorSubcoreMesh(core_axis_name="c", subcore_axis_name="s",
                                num_cores=NC),
    out_type=[
        jax.ShapeDtypeStruct((B, 128), jnp.float32),
        jax.ShapeDtypeStruct((B, 128), jnp.float32),
    ],
    scratch_types=[
        pltpu.VMEM((NCH, CH), jnp.int32),
        pltpu.VMEM((NCH, CH), jnp.int32),
        pltpu.VMEM((NCH, CH), jnp.int32),
        pltpu.VMEM((NCH, CH), jnp.int32),
        pltpu.VMEM((HB, 128), jnp.float32),
        pltpu.VMEM((HB, 128), jnp.float32),
        pltpu.SemaphoreType.DMA,
    ],
    compiler_params=pltpu.CompilerParams(use_tc_tiling_on_sc=True),
)(_gather_body)


SBLK = 2048


def _score_body(users_ref, items_ref, up_ref, ip_ref,
                o_ref, gu_ref, gi_ref):
    uhalf = (users_ref[...] & 1).astype(jnp.float32)[:, None]
    iq = items_ref[...] & 3
    up = up_ref[...]
    ip = ip_ref[...]
    gu = up[:, :DU] * (1.0 - uhalf) + up[:, DU:] * uhalf
    m0 = (iq == 0).astype(jnp.float32)[:, None]
    m1 = (iq == 1).astype(jnp.float32)[:, None]
    m2 = (iq == 2).astype(jnp.float32)[:, None]
    m3 = (iq == 3).astype(jnp.float32)[:, None]
    gi = (ip[:, :DI] * m0 + ip[:, DI:2 * DI] * m1
          + ip[:, 2 * DI:3 * DI] * m2 + ip[:, 3 * DI:] * m3)
    gu_ref[...] = gu
    gi_ref[...] = gi
    s0 = jnp.sum(gu[:, :DI] * gi, axis=1)
    s1 = jnp.sum(gu[:, DI:] * gi, axis=1)
    a0 = jax.nn.sigmoid(s0)
    a1 = jax.nn.sigmoid(s1)
    sum_log = jnp.log(1.0 - a0 + EPS) + jnp.log(1.0 - a1 + EPS)
    o_ref[...] = 1.0 - (-1.0 / (-1.0 + sum_log))


_score = pl.pallas_call(
    _score_body,
    grid=(B // SBLK,),
    in_specs=[pl.BlockSpec((SBLK,), lambda i: (i,)),
              pl.BlockSpec((SBLK,), lambda i: (i,)),
              pl.BlockSpec((SBLK, 128), lambda i: (i, 0)),
              pl.BlockSpec((SBLK, 128), lambda i: (i, 0))],
    out_specs=[pl.BlockSpec((SBLK,), lambda i: (i,)),
               pl.BlockSpec((SBLK, DU), lambda i: (i, 0)),
               pl.BlockSpec((SBLK, DI), lambda i: (i, 0))],
    out_shape=[jax.ShapeDtypeStruct((B,), jnp.float32),
               jax.ShapeDtypeStruct((B, DU), jnp.float32),
               jax.ShapeDtypeStruct((B, DI), jnp.float32)],
)


def kernel(users, items, Gu, Gi):
    users2 = users.reshape(NW * NCH, CH)
    items2 = items.reshape(NW * NCH, CH)
    gu_packed = Gu.reshape(1000000 * DU // 128, 128)
    gi_packed = Gi.reshape(100000 * DI // 128, 128)
    upair, ipair = _sc_gather(users2, items2, gu_packed, gi_packed)
    xui, gu_flat, gi_rows = _score(users, items, upair, ipair)
    return (xui, gu_flat.reshape(B, 2, DI), gi_rows)


# speedup vs baseline: 1.0158x; 1.0158x over previous
import functools

import jax
import jax.numpy as jnp
from jax import lax
from jax.experimental import pallas as pl
from jax.experimental.pallas import tpu as pltpu
from jax.experimental.pallas import tpu_sc as plsc

B = 16384
DU = 64
DI = 32
EPS = 1e-06

NC = 2
NS = 16
NW = NC * NS
BPW = B // NW
CH = 128
NCH = BPW // CH
HB = 256
L = 16


def _gather_body(users2, items2, gu_tab, gi_tab, gu_out, gi_out,
                 uidx, iidx, upidx, ipidx, upair, ipair, sem):
    wid = lax.axis_index("s") * NC + lax.axis_index("c")
    base = wid * BPW
    pltpu.sync_copy(users2.at[pl.ds(wid * NCH, NCH)], uidx)
    pltpu.sync_copy(items2.at[pl.ds(wid * NCH, NCH)], iidx)
    for c in range(NCH):
        for k in range(CH // L):
            s = pl.ds(k * L, L)
            upidx[c, s] = uidx[c, s] >> 1
            ipidx[c, s] = iidx[c, s] >> 2
    for h in range(BPW // HB):
        copies = []
        for c in range(2):
            cc = h * 2 + c
            copies.append(pltpu.async_copy(
                gu_tab.at[upidx.at[cc]], upair.at[pl.ds(c * CH, CH)], sem))
            copies.append(pltpu.async_copy(
                gi_tab.at[ipidx.at[cc]], ipair.at[pl.ds(c * CH, CH)], sem))
        for cp in copies:
            cp.wait()
        pltpu.sync_copy(upair, gu_out.at[pl.ds(base + h * HB, HB)])
        pltpu.sync_copy(ipair, gi_out.at[pl.ds(base + h * HB, HB)])


_sc_gather = functools.partial(
    pl.kernel,
    mesh=plsc.VectorSubcoreMesh(core_axis_name="c", subcore_axis_name="s",
                                num_cores=NC),
    out_type=[
        jax.ShapeDtypeStruct((B, 128), jnp.float32),
        jax.ShapeDtypeStruct((B, 128), jnp.float32),
    ],
    scratch_types=[
        pltpu.VMEM((NCH, CH), jnp.int32),
        pltpu.VMEM((NCH, CH), jnp.int32),
        pltpu.VMEM((NCH, CH), jnp.int32),
        pltpu.VMEM((NCH, CH), jnp.int32),
        pltpu.VMEM((HB, 128), jnp.float32),
        pltpu.VMEM((HB, 128), jnp.float32),
        pltpu.SemaphoreType.DMA,
    ],
    compiler_params=pltpu.CompilerParams(
        use_tc_tiling_on_sc=True, skip_device_barrier=True),
)(_gather_body)


SBLK = 2048


def _score_body(users_ref, items_ref, up_ref, ip_ref,
                o_ref, gu_ref, gi_ref):
    uhalf = (users_ref[...] & 1).astype(jnp.float32)[:, None]
    iq = items_ref[...] & 3
    up = up_ref[...]
    ip = ip_ref[...]
    gu = up[:, :DU] * (1.0 - uhalf) + up[:, DU:] * uhalf
    m0 = (iq == 0).astype(jnp.float32)[:, None]
    m1 = (iq == 1).astype(jnp.float32)[:, None]
    m2 = (iq == 2).astype(jnp.float32)[:, None]
    m3 = (iq == 3).astype(jnp.float32)[:, None]
    gi = (ip[:, :DI] * m0 + ip[:, DI:2 * DI] * m1
          + ip[:, 2 * DI:3 * DI] * m2 + ip[:, 3 * DI:] * m3)
    gu_ref[...] = gu
    gi_ref[...] = gi
    s0 = jnp.sum(gu[:, :DI] * gi, axis=1)
    s1 = jnp.sum(gu[:, DI:] * gi, axis=1)
    a0 = jax.nn.sigmoid(s0)
    a1 = jax.nn.sigmoid(s1)
    sum_log = jnp.log(1.0 - a0 + EPS) + jnp.log(1.0 - a1 + EPS)
    o_ref[...] = 1.0 - (-1.0 / (-1.0 + sum_log))


_score = pl.pallas_call(
    _score_body,
    grid=(B // SBLK,),
    in_specs=[pl.BlockSpec((SBLK,), lambda i: (i,)),
              pl.BlockSpec((SBLK,), lambda i: (i,)),
              pl.BlockSpec((SBLK, 128), lambda i: (i, 0)),
              pl.BlockSpec((SBLK, 128), lambda i: (i, 0))],
    out_specs=[pl.BlockSpec((SBLK,), lambda i: (i,)),
               pl.BlockSpec((SBLK, DU), lambda i: (i, 0)),
               pl.BlockSpec((SBLK, DI), lambda i: (i, 0))],
    out_shape=[jax.ShapeDtypeStruct((B,), jnp.float32),
               jax.ShapeDtypeStruct((B, DU), jnp.float32),
               jax.ShapeDtypeStruct((B, DI), jnp.float32)],
)


def kernel(users, items, Gu, Gi):
    users2 = users.reshape(NW * NCH, CH)
    items2 = items.reshape(NW * NCH, CH)
    gu_packed = Gu.reshape(1000000 * DU // 128, 128)
    gi_packed = Gi.reshape(100000 * DI // 128, 128)
    upair, ipair = _sc_gather(users2, items2, gu_packed, gi_packed)
    xui, gu_flat, gi_rows = _score(users, items, upair, ipair)
    return (xui, gu_flat.reshape(B, 2, DI), gi_rows)


# speedup vs baseline: 1.0418x; 1.0256x over previous
import functools

import jax
import jax.numpy as jnp
from jax import lax
from jax.experimental import pallas as pl
from jax.experimental.pallas import tpu as pltpu
from jax.experimental.pallas import tpu_sc as plsc

B = 16384
DU = 64
DI = 32
EPS = 1e-06

NC = 2
NS = 16
NW = NC * NS
BPW = B // NW
CH = 128
NCH = BPW // CH


def _gather_body(users2, items2, gu_tab, gi_tab, gu_out, gi_out,
                 uidx, iidx, urows, irows, sem):
    wid = lax.axis_index("s") * NC + lax.axis_index("c")
    base = wid * BPW
    pltpu.sync_copy(users2.at[pl.ds(wid * NCH, NCH)], uidx)
    pltpu.sync_copy(items2.at[pl.ds(wid * NCH, NCH)], iidx)
    copies = []
    for j in range(NCH):
        copies.append(pltpu.async_copy(
            gu_tab.at[uidx.at[j]], urows.at[pl.ds(j * CH, CH)], sem))
        copies.append(pltpu.async_copy(
            gi_tab.at[iidx.at[j]], irows.at[pl.ds(j * CH, CH)], sem))
    for c in copies:
        c.wait()
    pltpu.sync_copy(urows, gu_out.at[pl.ds(base, BPW)])
    pltpu.sync_copy(irows, gi_out.at[pl.ds(base, BPW)])


_sc_gather = functools.partial(
    pl.kernel,
    mesh=plsc.VectorSubcoreMesh(core_axis_name="c", subcore_axis_name="s"),
    out_type=[
        jax.ShapeDtypeStruct((B, DU), jnp.float32),
        jax.ShapeDtypeStruct((B, DI), jnp.float32),
    ],
    scratch_types=[
        pltpu.VMEM((NCH, CH), jnp.int32),
        pltpu.VMEM((NCH, CH), jnp.int32),
        pltpu.VMEM((BPW, DU), jnp.float32),
        pltpu.VMEM((BPW, DI), jnp.float32),
        pltpu.SemaphoreType.DMA,
    ],
    compiler_params=pltpu.CompilerParams(use_tc_tiling_on_sc=False),
)(_gather_body)


SBLK = 2048


def _score_body(gu_ref, gi_ref, o_ref):
    gu = gu_ref[...]
    gi = gi_ref[...]
    s0 = jnp.sum(gu[:, :DI] * gi, axis=1)
    s1 = jnp.sum(gu[:, DI:] * gi, axis=1)
    a0 = jax.nn.sigmoid(s0)
    a1 = jax.nn.sigmoid(s1)
    sum_log = jnp.log(1.0 - a0 + EPS) + jnp.log(1.0 - a1 + EPS)
    o_ref[...] = 1.0 - (-1.0 / (-1.0 + sum_log))


_score = pl.pallas_call(
    _score_body,
    grid=(B // SBLK,),
    in_specs=[pl.BlockSpec((SBLK, DU), lambda i: (i, 0)),
              pl.BlockSpec((SBLK, DI), lambda i: (i, 0))],
    out_specs=pl.BlockSpec((SBLK,), lambda i: (i,)),
    out_shape=jax.ShapeDtypeStruct((B,), jnp.float32),
)


def kernel(users, items, Gu, Gi):
    users2 = users.reshape(NW * NCH, CH)
    items2 = items.reshape(NW * NCH, CH)
    gu_flat, gi_rows = _sc_gather(users2, items2, Gu, Gi)
    xui = _score(gu_flat, gi_rows)
    return (xui, gu_flat.reshape(B, 2, DI), gi_rows)
